# baseline (speedup 1.0000x reference)
import jax
import jax.numpy as jnp
from jax import lax
from jax.experimental import pallas as pl
from jax.experimental.pallas import tpu as pltpu
from jax.experimental.pallas import tpu_sc as plsc

NUM_EMBEDDINGS = 1024
EMBEDDING_DIM = 32
COMMITMENT_COST = 0.25
TOPO_WEIGHT = 0.35
GRID_SIZE = 32
BATCH = 64
H = 32
W_SP = 32
PIX = H * W_SP
B_BLK = 8
PIXB = B_BLK * PIX
N_STEPS = BATCH // B_BLK
N_TOTAL = BATCH * PIX * EMBEDDING_DIM
EXT_ROWS = 8

NC = 2
NS = 16
LANES = 16
N_WORKERS = NC * NS
IMGS_PER_W = BATCH // N_WORKERS


def _vq_kernel(z_ref, w_ref, idx_ref, vq_ref, topo_ref, wn_ref,
               w2_ref, ext_ref, dacc_ref):
    pid = pl.program_id(0)

    @pl.when(pid == 0)
    def _prologue():
        w = w_ref[...]
        w_norm2 = jnp.sum(w * w, axis=1, keepdims=True)
        w_n = w / jnp.maximum(jnp.sqrt(w_norm2), 1e-12)
        wn_ref[...] = w_n
        w2_ref[...] = -2.0 * w_n
        ids = lax.broadcasted_iota(jnp.int32, (1, NUM_EMBEDDINGS), 1)
        ext_ref[0:1, :] = jnp.ones((1, NUM_EMBEDDINGS), jnp.float32)
        ext_ref[1:2, :] = (ids // 8).astype(jnp.float32)
        ext_ref[2:3, :] = (ids % 8).astype(jnp.float32)
        ext_ref[3:8, :] = jnp.zeros((5, NUM_EMBEDDINGS), jnp.float32)
        dacc_ref[...] = jnp.zeros((1, PIXB), jnp.float32)
        g = w_n.reshape(GRID_SIZE, GRID_SIZE, EMBEDDING_DIM)
        dh = g[:, 1:, :] - g[:, :-1, :]
        dv = g[1:, :, :] - g[:-1, :, :]
        wh = g[:, 0, :] - g[:, -1, :]
        wv = g[0, :, :] - g[-1, :, :]
        t = (jnp.sum(dh * dh) / dh.size + jnp.sum(dv * dv) / dv.size
             + jnp.sum(wh * wh) / wh.size + jnp.sum(wv * wv) / wv.size)
        topo_ref[...] = (TOPO_WEIGHT * t).reshape(1, 1)

    z = jnp.concatenate([z_ref[b] for b in range(B_BLK)], axis=1)
    z_norm2 = jnp.sum(z * z, axis=0, keepdims=True)
    z_n = z / jnp.maximum(jnp.sqrt(z_norm2), 1e-12)

    d = 2.0 + jnp.dot(w2_ref[...], z_n, preferred_element_type=jnp.float32)
    d_min = jnp.min(d, axis=0, keepdims=True)
    onehot = (d == d_min).astype(jnp.float32)

    acc = jnp.dot(ext_ref[...], onehot, preferred_element_type=jnp.float32)
    inv = 1.0 / acc[0:1, :]
    idx = ((8.0 * acc[1:2, :] + acc[2:3, :]) * inv).astype(jnp.int32)

    for b in range(B_BLK):
        idx_ref[b] = idx[:, b * PIX:(b + 1) * PIX]

    dacc_ref[...] += d_min

    @pl.when(pid == N_STEPS - 1)
    def _epilogue():
        sq = jnp.sum(dacc_ref[...])
        vq_ref[...] = (sq * ((1.0 + COMMITMENT_COST) / N_TOTAL)).reshape(1, 1)


def _sc_gather(w_hbm, idx_hbm, zq_hbm, table_v, idx_v, out_v):
    c = lax.axis_index("c")
    s = lax.axis_index("s")
    wid = s * NC + c
    pltpu.sync_copy(w_hbm, table_v)
    for k in range(IMGS_PER_W):
        b = wid * IMGS_PER_W + k
        pltpu.sync_copy(idx_hbm.at[b], idx_v)

        def chunk_body(i, carry):
            pi = idx_v[pl.ds(i * LANES, LANES)]
            base = pi * EMBEDDING_DIM
            for dd in range(EMBEDDING_DIM):
                out_v[dd, pl.ds(i * LANES, LANES)] = plsc.load_gather(
                    table_v, [base + dd])
            return carry

        lax.fori_loop(0, PIX // LANES, chunk_body, 0)
        pltpu.sync_copy(out_v, zq_hbm.at[b])


def kernel(z_e, W):
    z_flat = z_e.reshape(BATCH, EMBEDDING_DIM, PIX)
    idx, vq, topo, w_n = pl.pallas_call(
        _vq_kernel,
        grid=(N_STEPS,),
        in_specs=[
            pl.BlockSpec((B_BLK, EMBEDDING_DIM, PIX), lambda i: (i, 0, 0)),
            pl.BlockSpec((NUM_EMBEDDINGS, EMBEDDING_DIM), lambda i: (0, 0)),
        ],
        out_specs=[
            pl.BlockSpec((B_BLK, 1, PIX), lambda i: (i, 0, 0)),
            pl.BlockSpec((1, 1), lambda i: (0, 0)),
            pl.BlockSpec((1, 1), lambda i: (0, 0)),
            pl.BlockSpec((NUM_EMBEDDINGS, EMBEDDING_DIM), lambda i: (0, 0)),
        ],
        out_shape=[
            jax.ShapeDtypeStruct((BATCH, 1, PIX), jnp.int32),
            jax.ShapeDtypeStruct((1, 1), jnp.float32),
            jax.ShapeDtypeStruct((1, 1), jnp.float32),
            jax.ShapeDtypeStruct((NUM_EMBEDDINGS, EMBEDDING_DIM), jnp.float32),
        ],
        scratch_shapes=[
            pltpu.VMEM((NUM_EMBEDDINGS, EMBEDDING_DIM), jnp.float32),
            pltpu.VMEM((EXT_ROWS, NUM_EMBEDDINGS), jnp.float32),
            pltpu.VMEM((1, PIXB), jnp.float32),
        ],
    )(z_flat, W)

    idx2 = idx.reshape(BATCH, PIX)
    mesh = plsc.VectorSubcoreMesh(core_axis_name="c", subcore_axis_name="s")
    zq = pl.kernel(
        _sc_gather,
        out_type=jax.ShapeDtypeStruct((BATCH, EMBEDDING_DIM, PIX),
                                      jnp.float32),
        mesh=mesh,
        compiler_params=pltpu.CompilerParams(needs_layout_passes=False),
        scratch_types=[
            pltpu.VMEM((NUM_EMBEDDINGS * EMBEDDING_DIM,), jnp.float32),
            pltpu.VMEM((PIX,), jnp.int32),
            pltpu.VMEM((EMBEDDING_DIM, PIX), jnp.float32),
        ],
    )(w_n.reshape(NUM_EMBEDDINGS * EMBEDDING_DIM), idx2)

    return (zq.reshape(BATCH, EMBEDDING_DIM, H, W_SP), vq.reshape(()),
            idx.reshape(BATCH, H, W_SP), topo.reshape(()))


# speedup vs baseline: 1.7450x; 1.7450x over previous
import jax
import jax.numpy as jnp
from jax import lax
from jax.experimental import pallas as pl
from jax.experimental.pallas import tpu as pltpu

NUM_EMBEDDINGS = 1024
EMBEDDING_DIM = 32
COMMITMENT_COST = 0.25
TOPO_WEIGHT = 0.35
GRID_SIZE = 32
BATCH = 64
H = 32
W_SP = 32
PIX = H * W_SP
B_BLK = 8
PIXB = B_BLK * PIX
N_STEPS = BATCH // B_BLK
N_TOTAL = BATCH * PIX * EMBEDDING_DIM
EXT_ROWS = 8 + EMBEDDING_DIM


def _vq_kernel(z_ref, w_ref, zq_ref, idx_ref, vq_ref, topo_ref,
               w2_ref, ext_ref, dacc_ref):
    pid = pl.program_id(0)

    @pl.when(pid == 0)
    def _prologue():
        w = w_ref[...]
        w_norm2 = jnp.sum(w * w, axis=1, keepdims=True)
        w_n = w / jnp.maximum(jnp.sqrt(w_norm2), 1e-12)
        w2_ref[...] = -2.0 * w_n
        ids = lax.broadcasted_iota(jnp.int32, (1, NUM_EMBEDDINGS), 1)
        ext_ref[0:1, :] = jnp.ones((1, NUM_EMBEDDINGS), jnp.float32)
        ext_ref[1:2, :] = ids.astype(jnp.float32)
        ext_ref[2:8, :] = jnp.zeros((6, NUM_EMBEDDINGS), jnp.float32)
        ext_ref[8:EXT_ROWS, :] = w_n.T
        dacc_ref[...] = jnp.zeros((1, PIXB), jnp.float32)
        g = w_n.reshape(GRID_SIZE, GRID_SIZE, EMBEDDING_DIM)
        dh = g[:, 1:, :] - g[:, :-1, :]
        dv = g[1:, :, :] - g[:-1, :, :]
        wh = g[:, 0, :] - g[:, -1, :]
        wv = g[0, :, :] - g[-1, :, :]
        t = (jnp.sum(dh * dh) / dh.size + jnp.sum(dv * dv) / dv.size
             + jnp.sum(wh * wh) / wh.size + jnp.sum(wv * wv) / wv.size)
        topo_ref[...] = (TOPO_WEIGHT * t).reshape(1, 1)

    z = jnp.concatenate([z_ref[b] for b in range(B_BLK)], axis=1)
    z_norm2 = jnp.sum(z * z, axis=0, keepdims=True)
    z_n = z / jnp.maximum(jnp.sqrt(z_norm2), 1e-12)

    d = 2.0 + jnp.dot(w2_ref[...], z_n, preferred_element_type=jnp.float32)
    d_min = jnp.min(d, axis=0, keepdims=True)
    onehot = (d == d_min).astype(jnp.float32)

    acc = jnp.dot(ext_ref[...], onehot, preferred_element_type=jnp.float32)
    inv = 1.0 / acc[0:1, :]
    idx = (acc[1:2, :] * inv).astype(jnp.int32)
    z_q = acc[8:EXT_ROWS, :] * inv

    for b in range(B_BLK):
        idx_ref[b] = idx[:, b * PIX:(b + 1) * PIX]
        zq_ref[b] = z_q[:, b * PIX:(b + 1) * PIX]

    dacc_ref[...] += d_min

    @pl.when(pid == N_STEPS - 1)
    def _epilogue():
        sq = jnp.sum(dacc_ref[...])
        vq_ref[...] = (sq * ((1.0 + COMMITMENT_COST) / N_TOTAL)).reshape(1, 1)


def kernel(z_e, W):
    z_flat = z_e.reshape(BATCH, EMBEDDING_DIM, PIX)
    zq, idx, vq, topo = pl.pallas_call(
        _vq_kernel,
        grid=(N_STEPS,),
        in_specs=[
            pl.BlockSpec((B_BLK, EMBEDDING_DIM, PIX), lambda i: (i, 0, 0)),
            pl.BlockSpec((NUM_EMBEDDINGS, EMBEDDING_DIM), lambda i: (0, 0)),
        ],
        out_specs=[
            pl.BlockSpec((B_BLK, EMBEDDING_DIM, PIX), lambda i: (i, 0, 0)),
            pl.BlockSpec((B_BLK, 1, PIX), lambda i: (i, 0, 0)),
            pl.BlockSpec((1, 1), lambda i: (0, 0)),
            pl.BlockSpec((1, 1), lambda i: (0, 0)),
        ],
        out_shape=[
            jax.ShapeDtypeStruct((BATCH, EMBEDDING_DIM, PIX), jnp.float32),
            jax.ShapeDtypeStruct((BATCH, 1, PIX), jnp.int32),
            jax.ShapeDtypeStruct((1, 1), jnp.float32),
            jax.ShapeDtypeStruct((1, 1), jnp.float32),
        ],
        scratch_shapes=[
            pltpu.VMEM((NUM_EMBEDDINGS, EMBEDDING_DIM), jnp.float32),
            pltpu.VMEM((EXT_ROWS, NUM_EMBEDDINGS), jnp.float32),
            pltpu.VMEM((1, PIXB), jnp.float32),
        ],
    )(z_flat, W)
    return (zq.reshape(BATCH, EMBEDDING_DIM, H, W_SP), vq.reshape(()),
            idx.reshape(BATCH, H, W_SP), topo.reshape(()))
